# baseline (speedup 1.0000x reference)
import functools

import numpy as np
import jax
import jax.numpy as jnp
from jax import lax
from jax.experimental import pallas as pl
from jax.experimental.pallas import tpu as pltpu
from jax.experimental.pallas import tpu_sc as plsc

_N = 10000
_E = 320000
_D = 128
_H = 128
_DENSE = 64
_NCLS = 2
_NG = 128
_EPS = 1e-5

_NC = 2
_NS = 16
_NW = _NC * _NS
_K = 64
_CHD = 125
_KD = 80
_EPW = 10240
_CH = _EPW // _K
_EPAD = _NW * _EPW - _E
_NP = _N + 8
_RPT = _N // _NS

_BM = 1000
_G = _N // _BM

_BNS = np.float32(1.0 / np.sqrt(1.0 + _EPS))


def _sc_mesh():
    return plsc.VectorSubcoreMesh(core_axis_name="c", subcore_axis_name="s")


def _deg_partial(dst3, onesrow, zrows):
    @functools.partial(
        pl.kernel,
        out_type=jax.ShapeDtypeStruct((_NC, _NS, _RPT, 16), jnp.float32),
        mesh=_sc_mesh(),
        scratch_types=[
            pltpu.VMEM((_CHD, _KD), jnp.int32),
            pltpu.VMEM((_KD, 16), jnp.float32),
            pltpu.VMEM_SHARED((_N, 16), jnp.float32),
        ],
    )
    def k(dst_hbm, ones_hbm, z_hbm, out_hbm, dstv, ones, acc):
        c = lax.axis_index("c")
        s = lax.axis_index("s")
        tid = c * _NS + s
        pltpu.sync_copy(dst_hbm.at[tid], dstv)
        pltpu.sync_copy(ones_hbm, ones)
        base = s * _RPT
        pltpu.sync_copy(z_hbm.at[s], acc.at[pl.ds(base, _RPT)])
        plsc.subcore_barrier()

        @pl.loop(0, _CHD)
        def _(ci):
            pltpu.sync_copy(ones, acc.at[dstv.at[ci]], add=True)

        plsc.subcore_barrier()
        pltpu.sync_copy(acc.at[pl.ds(base, _RPT)], out_hbm.at[c, s])

    return k(dst3, onesrow, zrows)


def _prop_partial(hs, packed3, zblk):
    def unpack(pk_row, srcb, dstb):
        for j in range(_K // 16):
            pk = pk_row[pl.ds(j * 16, 16)]
            srcb[pl.ds(j * 16, 16)] = lax.bitwise_and(pk, 0xFFFF)
            dstb[pl.ds(j * 16, 16)] = lax.shift_right_logical(pk, 16)

    @functools.partial(
        pl.kernel,
        out_type=jax.ShapeDtypeStruct((_NC, _NS, _RPT, _H), jnp.float32),
        mesh=_sc_mesh(),
        scratch_types=[
            pltpu.VMEM((_CH, _K), jnp.int32),
            pltpu.VMEM((_K,), jnp.int32),
            pltpu.VMEM((_K,), jnp.int32),
            pltpu.VMEM((_K,), jnp.int32),
            pltpu.VMEM((_K,), jnp.int32),
            pltpu.VMEM((_K, _H), jnp.float32),
            pltpu.VMEM((_K, _H), jnp.float32),
            pltpu.VMEM_SHARED((_NP, _H), jnp.float32),
            pltpu.SemaphoreType.DMA,
            pltpu.SemaphoreType.DMA,
        ],
    )
    def k(hs_hbm, pk_hbm, z_hbm, out_hbm,
          pkv, src_a, dst_a, src_b, dst_b, rows_a, rows_b, acc, sem_a, sem_b):
        c = lax.axis_index("c")
        s = lax.axis_index("s")
        tid = c * _NS + s
        pltpu.sync_copy(pk_hbm.at[tid], pkv)
        base = s * _RPT
        pltpu.sync_copy(z_hbm.at[s], acc.at[pl.ds(base, _RPT)])
        plsc.subcore_barrier()

        @pl.loop(0, _CH, step=2)
        def _(ci):
            @pl.when(ci >= 2)
            def _():
                pltpu.make_async_copy(rows_a, acc.at[dst_a], sem_a).wait()

            unpack(pkv.at[ci], src_a, dst_a)
            pltpu.sync_copy(hs_hbm.at[src_a], rows_a)
            pltpu.async_copy(rows_a, acc.at[dst_a], sem_a, add=True)

            @pl.when(ci >= 2)
            def _():
                pltpu.make_async_copy(rows_b, acc.at[dst_b], sem_b).wait()

            unpack(pkv.at[ci + 1], src_b, dst_b)
            pltpu.sync_copy(hs_hbm.at[src_b], rows_b)
            pltpu.async_copy(rows_b, acc.at[dst_b], sem_b, add=True)

        pltpu.make_async_copy(rows_a, acc.at[dst_a], sem_a).wait()
        pltpu.make_async_copy(rows_b, acc.at[dst_b], sem_b).wait()
        plsc.subcore_barrier()
        pltpu.sync_copy(acc.at[pl.ds(base, _RPT)], out_hbm.at[c, s])

    return k(hs, packed3, zblk)


def _matmul(x, W):
    def body(x_ref, w_ref, o_ref):
        o_ref[...] = jnp.dot(x_ref[...], w_ref[...],
                             preferred_element_type=jnp.float32)

    return pl.pallas_call(
        body,
        grid=(_G,),
        in_specs=[
            pl.BlockSpec((_BM, _D), lambda i: (i, 0)),
            pl.BlockSpec((_D, _H), lambda i: (0, 0)),
        ],
        out_specs=pl.BlockSpec((_BM, _H), lambda i: (i, 0)),
        out_shape=jax.ShapeDtypeStruct((_N, _H), jnp.float32),
    )(x, W)


def _dinv_of(degp_ref):
    deg = degp_ref[0, :, 0:1] + degp_ref[1, :, 0:1] + 1.0
    return lax.rsqrt(deg)


def _scale(h, degp):
    def body(h_ref, degp_ref, o_ref):
        o_ref[...] = h_ref[...] * _dinv_of(degp_ref)

    return pl.pallas_call(
        body,
        grid=(_G,),
        in_specs=[
            pl.BlockSpec((_BM, _H), lambda i: (i, 0)),
            pl.BlockSpec((_NC, _BM, 16), lambda i: (0, i, 0)),
        ],
        out_specs=pl.BlockSpec((_BM, _H), lambda i: (i, 0)),
        out_shape=jax.ShapeDtypeStruct((_N, _H), jnp.float32),
    )(h, degp)


def _layer2(p, hs1, degp, b1r, W2):
    def body(p_ref, hs1_ref, degp_ref, b1_ref, w2_ref, o_ref):
        dinv = _dinv_of(degp_ref)
        h1 = jnp.maximum(
            (p_ref[0] + p_ref[1] + hs1_ref[...]) * dinv + b1_ref[...], 0.0)
        o_ref[...] = jnp.dot(h1, w2_ref[...],
                             preferred_element_type=jnp.float32) * dinv

    return pl.pallas_call(
        body,
        grid=(_G,),
        in_specs=[
            pl.BlockSpec((_NC, _BM, _H), lambda i: (0, i, 0)),
            pl.BlockSpec((_BM, _H), lambda i: (i, 0)),
            pl.BlockSpec((_NC, _BM, 16), lambda i: (0, i, 0)),
            pl.BlockSpec((1, _H), lambda i: (0, 0)),
            pl.BlockSpec((_H, _H), lambda i: (0, 0)),
        ],
        out_specs=pl.BlockSpec((_BM, _H), lambda i: (i, 0)),
        out_shape=jax.ShapeDtypeStruct((_N, _H), jnp.float32),
    )(p, hs1, degp, b1r, W2)


def _final(q, hs2, degp, batch2, b2r, bngr, bnbr, bn1gr, bn1br,
           l1W, l1br, l2W, l2br, l3W, l3br):
    def body(q_ref, hs2_ref, degp_ref, batch_ref, b2_ref, bng_ref, bnb_ref,
             bn1g_ref, bn1b_ref, l1w_ref, l1b_ref, l2w_ref, l2b_ref,
             l3w_ref, l3b_ref, out_ref, xp_ref, sums_ref, cnts_ref):
        i = pl.program_id(0)

        @pl.when(i == 0)
        def _():
            sums_ref[...] = jnp.zeros_like(sums_ref)
            cnts_ref[...] = jnp.zeros_like(cnts_ref)

        dinv = _dinv_of(degp_ref)
        pre = (q_ref[0] + q_ref[1] + hs2_ref[...]) * dinv + b2_ref[...]
        h2 = jnp.maximum(pre * (bng_ref[...] * _BNS) + bnb_ref[...], 0.0)
        oh = (batch_ref[...] ==
              lax.broadcasted_iota(jnp.int32, (1, _NG), 1)).astype(jnp.float32)
        sums_ref[...] += lax.dot_general(
            oh, h2, (((0,), (0,)), ((), ())),
            preferred_element_type=jnp.float32)
        cnts_ref[...] += lax.dot_general(
            oh, jnp.ones((_BM, 8), jnp.float32), (((0,), (0,)), ((), ())),
            preferred_element_type=jnp.float32)

        @pl.when(i == _G - 1)
        def _():
            xp = sums_ref[...] / jnp.maximum(cnts_ref[:, 0:1], 1.0)
            xb = xp * (bn1g_ref[...] * _BNS) + bn1b_ref[...]
            a = jnp.maximum(
                jnp.dot(xb, l1w_ref[...],
                        preferred_element_type=jnp.float32) + l1b_ref[...], 0.0)
            a = jnp.maximum(
                jnp.dot(a, l2w_ref[...],
                        preferred_element_type=jnp.float32) + l2b_ref[...], 0.0)
            a = jnp.maximum(
                jnp.dot(a, l3w_ref[...],
                        preferred_element_type=jnp.float32) + l3b_ref[...], 0.0)
            m = jnp.max(a, axis=1, keepdims=True)
            e = jnp.exp(a - m)
            out_ref[...] = e / jnp.sum(e, axis=1, keepdims=True)
            xp_ref[...] = xp

    full = lambda i: (0, 0)
    return pl.pallas_call(
        body,
        grid=(_G,),
        in_specs=[
            pl.BlockSpec((_NC, _BM, _H), lambda i: (0, i, 0)),
            pl.BlockSpec((_BM, _H), lambda i: (i, 0)),
            pl.BlockSpec((_NC, _BM, 16), lambda i: (0, i, 0)),
            pl.BlockSpec((_BM, 1), lambda i: (i, 0)),
            pl.BlockSpec((1, _H), full),
            pl.BlockSpec((1, _H), full),
            pl.BlockSpec((1, _H), full),
            pl.BlockSpec((1, _H), full),
            pl.BlockSpec((1, _H), full),
            pl.BlockSpec((_H, _DENSE), full),
            pl.BlockSpec((1, _DENSE), full),
            pl.BlockSpec((_DENSE, _DENSE), full),
            pl.BlockSpec((1, _DENSE), full),
            pl.BlockSpec((_DENSE, _NCLS), full),
            pl.BlockSpec((1, _NCLS), full),
        ],
        out_specs=[
            pl.BlockSpec((_NG, _NCLS), full),
            pl.BlockSpec((_NG, _H), full),
        ],
        out_shape=[
            jax.ShapeDtypeStruct((_NG, _NCLS), jnp.float32),
            jax.ShapeDtypeStruct((_NG, _H), jnp.float32),
        ],
        scratch_shapes=[
            pltpu.VMEM((_NG, _H), jnp.float32),
            pltpu.VMEM((_NG, 8), jnp.float32),
        ],
    )(q, hs2, degp, batch2, b2r, bngr, bnbr, bn1gr, bn1br,
      l1W, l1br, l2W, l2br, l3W, l3br)


def kernel(x, edge_index, batch, W1, b1, W2, b2, bng, bnb, bn1g, bn1b,
           l1W, l1b, l2W, l2b, l3W, l3b):
    src = edge_index[0].astype(jnp.int32)
    dst = edge_index[1].astype(jnp.int32)
    dst3 = dst.reshape(_NW, _CHD, _KD)
    pad = jnp.full((_EPAD,), _N << 16, jnp.int32)
    packed3 = jnp.concatenate(
        [src | (dst << 16), pad]).reshape(_NW, _CH, _K)
    batch2 = batch.astype(jnp.int32).reshape(_N, 1)

    onesrow = jnp.concatenate(
        [jnp.ones((_KD, 1), jnp.float32), jnp.zeros((_KD, 15), jnp.float32)],
        axis=1)
    zrows = jnp.zeros((_NS, _RPT, 16), jnp.float32)
    zblk = jnp.zeros((_NS, _RPT, _H), jnp.float32)

    degp = _deg_partial(dst3, onesrow, zrows).reshape(_NC, _N, 16)
    h0 = _matmul(x, W1)
    hs1 = _scale(h0, degp)
    p = _prop_partial(hs1, packed3, zblk).reshape(_NC, _N, _H)
    hs2 = _layer2(p, hs1, degp, b1.reshape(1, _H), W2)
    q = _prop_partial(hs2, packed3, zblk).reshape(_NC, _N, _H)
    out, xp = _final(
        q, hs2, degp, batch2,
        b2.reshape(1, _H), bng.reshape(1, _H), bnb.reshape(1, _H),
        bn1g.reshape(1, _H), bn1b.reshape(1, _H),
        l1W, l1b.reshape(1, _DENSE), l2W, l2b.reshape(1, _DENSE),
        l3W, l3b.reshape(1, _NCLS))
    return (out, xp)


# speedup vs baseline: 2.7482x; 2.7482x over previous
import functools

import numpy as np
import jax
import jax.numpy as jnp
from jax import lax
from jax.experimental import pallas as pl
from jax.experimental.pallas import tpu as pltpu
from jax.experimental.pallas import tpu_sc as plsc

_N = 10000
_E = 320000
_D = 128
_H = 128
_DENSE = 64
_NCLS = 2
_NG = 128
_EPS = 1e-5

_NC = 2
_NS = 16
_NW = _NC * _NS
_K = 80
_CH = _E // (_NW * _K)
_CHD = 80
_KD = 125
_RPT = _N // _NS

_BM = 1000
_G = _N // _BM

_BNS = np.float32(1.0 / np.sqrt(1.0 + _EPS))


def _sc_mesh():
    return plsc.VectorSubcoreMesh(core_axis_name="c", subcore_axis_name="s")


def _deg_partial(dst3, onesrow, zrows):
    @functools.partial(
        pl.kernel,
        out_type=jax.ShapeDtypeStruct((_NC, _NS, _RPT, 16), jnp.float32),
        mesh=_sc_mesh(),
        scratch_types=[
            pltpu.VMEM((_CHD, _KD), jnp.int32),
            pltpu.VMEM((_KD, 16), jnp.float32),
            pltpu.VMEM_SHARED((_N, 16), jnp.float32),
        ],
    )
    def k(dst_hbm, ones_hbm, z_hbm, out_hbm, dstv, ones, acc):
        c = lax.axis_index("c")
        s = lax.axis_index("s")
        tid = c * _NS + s
        pltpu.sync_copy(dst_hbm.at[tid], dstv)
        pltpu.sync_copy(ones_hbm, ones)
        base = s * _RPT
        pltpu.sync_copy(z_hbm.at[s], acc.at[pl.ds(base, _RPT)])
        plsc.subcore_barrier()

        @pl.loop(0, _CHD)
        def _(ci):
            pltpu.sync_copy(ones, acc.at[dstv.at[ci]], add=True)

        plsc.subcore_barrier()
        pltpu.sync_copy(acc.at[pl.ds(base, _RPT)], out_hbm.at[c, s])

    return k(dst3, onesrow, zrows)


def _prop_partial(hs, packed3, zblk):
    def unpack(pk_row, srcb, dstb):
        for j in range(_K // 16):
            pk = pk_row[pl.ds(j * 16, 16)]
            srcb[pl.ds(j * 16, 16)] = lax.bitwise_and(pk, 0xFFFF)
            dstb[pl.ds(j * 16, 16)] = lax.shift_right_logical(pk, 16)

    @functools.partial(
        pl.kernel,
        out_type=jax.ShapeDtypeStruct((_NC, _NS, _RPT, _H), jnp.float32),
        mesh=_sc_mesh(),
        scratch_types=[
            pltpu.VMEM((_CH, _K), jnp.int32),
            pltpu.VMEM((_K,), jnp.int32),
            pltpu.VMEM((_K,), jnp.int32),
            pltpu.VMEM((_K,), jnp.int32),
            pltpu.VMEM((_K,), jnp.int32),
            pltpu.VMEM((_K, _H), jnp.float32),
            pltpu.VMEM((_K, _H), jnp.float32),
            pltpu.VMEM_SHARED((_N, _H), jnp.float32),
            pltpu.SemaphoreType.DMA,
            pltpu.SemaphoreType.DMA,
        ],
    )
    def k(hs_hbm, pk_hbm, z_hbm, out_hbm,
          pkv, src_a, dst_a, src_b, dst_b, rows_a, rows_b, acc, sem_a, sem_b):
        c = lax.axis_index("c")
        s = lax.axis_index("s")
        tid = c * _NS + s
        pltpu.sync_copy(pk_hbm.at[tid], pkv)
        base = s * _RPT
        pltpu.sync_copy(z_hbm.at[s], acc.at[pl.ds(base, _RPT)])
        plsc.subcore_barrier()

        @pl.loop(0, _CH - 1, step=2)
        def _(ci):
            @pl.when(ci >= 2)
            def _():
                pltpu.make_async_copy(rows_a, acc.at[dst_a], sem_a).wait()

            unpack(pkv.at[ci], src_a, dst_a)
            pltpu.sync_copy(hs_hbm.at[src_a], rows_a)
            pltpu.async_copy(rows_a, acc.at[dst_a], sem_a, add=True)

            @pl.when(ci >= 2)
            def _():
                pltpu.make_async_copy(rows_b, acc.at[dst_b], sem_b).wait()

            unpack(pkv.at[ci + 1], src_b, dst_b)
            pltpu.sync_copy(hs_hbm.at[src_b], rows_b)
            pltpu.async_copy(rows_b, acc.at[dst_b], sem_b, add=True)

        pltpu.make_async_copy(rows_a, acc.at[dst_a], sem_a).wait()
        unpack(pkv.at[_CH - 1], src_a, dst_a)
        pltpu.sync_copy(hs_hbm.at[src_a], rows_a)
        pltpu.async_copy(rows_a, acc.at[dst_a], sem_a, add=True)

        pltpu.make_async_copy(rows_a, acc.at[dst_a], sem_a).wait()
        pltpu.make_async_copy(rows_b, acc.at[dst_b], sem_b).wait()
        plsc.subcore_barrier()
        pltpu.sync_copy(acc.at[pl.ds(base, _RPT)], out_hbm.at[c, s])

    return k(hs, packed3, zblk)


def _matmul(x, W):
    def body(x_ref, w_ref, o_ref):
        o_ref[...] = jnp.dot(x_ref[...], w_ref[...],
                             preferred_element_type=jnp.float32)

    return pl.pallas_call(
        body,
        grid=(_G,),
        in_specs=[
            pl.BlockSpec((_BM, _D), lambda i: (i, 0)),
            pl.BlockSpec((_D, _H), lambda i: (0, 0)),
        ],
        out_specs=pl.BlockSpec((_BM, _H), lambda i: (i, 0)),
        out_shape=jax.ShapeDtypeStruct((_N, _H), jnp.float32),
    )(x, W)


def _dinv_of(degp_ref):
    deg = degp_ref[0, :, 0:1] + degp_ref[1, :, 0:1] + 1.0
    return lax.rsqrt(deg)


def _scale(h, degp):
    def body(h_ref, degp_ref, o_ref):
        o_ref[...] = h_ref[...] * _dinv_of(degp_ref)

    return pl.pallas_call(
        body,
        grid=(_G,),
        in_specs=[
            pl.BlockSpec((_BM, _H), lambda i: (i, 0)),
            pl.BlockSpec((_NC, _BM, 16), lambda i: (0, i, 0)),
        ],
        out_specs=pl.BlockSpec((_BM, _H), lambda i: (i, 0)),
        out_shape=jax.ShapeDtypeStruct((_N, _H), jnp.float32),
    )(h, degp)


def _layer2(p, hs1, degp, b1r, W2):
    def body(p_ref, hs1_ref, degp_ref, b1_ref, w2_ref, o_ref):
        dinv = _dinv_of(degp_ref)
        h1 = jnp.maximum(
            (p_ref[0] + p_ref[1] + hs1_ref[...]) * dinv + b1_ref[...], 0.0)
        o_ref[...] = jnp.dot(h1, w2_ref[...],
                             preferred_element_type=jnp.float32) * dinv

    return pl.pallas_call(
        body,
        grid=(_G,),
        in_specs=[
            pl.BlockSpec((_NC, _BM, _H), lambda i: (0, i, 0)),
            pl.BlockSpec((_BM, _H), lambda i: (i, 0)),
            pl.BlockSpec((_NC, _BM, 16), lambda i: (0, i, 0)),
            pl.BlockSpec((1, _H), lambda i: (0, 0)),
            pl.BlockSpec((_H, _H), lambda i: (0, 0)),
        ],
        out_specs=pl.BlockSpec((_BM, _H), lambda i: (i, 0)),
        out_shape=jax.ShapeDtypeStruct((_N, _H), jnp.float32),
    )(p, hs1, degp, b1r, W2)


def _final(q, hs2, degp, batch2, b2r, bngr, bnbr, bn1gr, bn1br,
           l1W, l1br, l2W, l2br, l3W, l3br):
    def body(q_ref, hs2_ref, degp_ref, batch_ref, b2_ref, bng_ref, bnb_ref,
             bn1g_ref, bn1b_ref, l1w_ref, l1b_ref, l2w_ref, l2b_ref,
             l3w_ref, l3b_ref, out_ref, xp_ref, sums_ref, cnts_ref):
        i = pl.program_id(0)

        @pl.when(i == 0)
        def _():
            sums_ref[...] = jnp.zeros_like(sums_ref)
            cnts_ref[...] = jnp.zeros_like(cnts_ref)

        dinv = _dinv_of(degp_ref)
        pre = (q_ref[0] + q_ref[1] + hs2_ref[...]) * dinv + b2_ref[...]
        h2 = jnp.maximum(pre * (bng_ref[...] * _BNS) + bnb_ref[...], 0.0)
        oh = (batch_ref[...] ==
              lax.broadcasted_iota(jnp.int32, (1, _NG), 1)).astype(jnp.float32)
        sums_ref[...] += lax.dot_general(
            oh, h2, (((0,), (0,)), ((), ())),
            preferred_element_type=jnp.float32)
        cnts_ref[...] += lax.dot_general(
            oh, jnp.ones((_BM, 8), jnp.float32), (((0,), (0,)), ((), ())),
            preferred_element_type=jnp.float32)

        @pl.when(i == _G - 1)
        def _():
            xp = sums_ref[...] / jnp.maximum(cnts_ref[:, 0:1], 1.0)
            xb = xp * (bn1g_ref[...] * _BNS) + bn1b_ref[...]
            a = jnp.maximum(
                jnp.dot(xb, l1w_ref[...],
                        preferred_element_type=jnp.float32) + l1b_ref[...], 0.0)
            a = jnp.maximum(
                jnp.dot(a, l2w_ref[...],
                        preferred_element_type=jnp.float32) + l2b_ref[...], 0.0)
            a = jnp.maximum(
                jnp.dot(a, l3w_ref[...],
                        preferred_element_type=jnp.float32) + l3b_ref[...], 0.0)
            m = jnp.max(a, axis=1, keepdims=True)
            e = jnp.exp(a - m)
            out_ref[...] = e / jnp.sum(e, axis=1, keepdims=True)
            xp_ref[...] = xp

    full = lambda i: (0, 0)
    return pl.pallas_call(
        body,
        grid=(_G,),
        in_specs=[
            pl.BlockSpec((_NC, _BM, _H), lambda i: (0, i, 0)),
            pl.BlockSpec((_BM, _H), lambda i: (i, 0)),
            pl.BlockSpec((_NC, _BM, 16), lambda i: (0, i, 0)),
            pl.BlockSpec((_BM, 1), lambda i: (i, 0)),
            pl.BlockSpec((1, _H), full),
            pl.BlockSpec((1, _H), full),
            pl.BlockSpec((1, _H), full),
            pl.BlockSpec((1, _H), full),
            pl.BlockSpec((1, _H), full),
            pl.BlockSpec((_H, _DENSE), full),
            pl.BlockSpec((1, _DENSE), full),
            pl.BlockSpec((_DENSE, _DENSE), full),
            pl.BlockSpec((1, _DENSE), full),
            pl.BlockSpec((_DENSE, _NCLS), full),
            pl.BlockSpec((1, _NCLS), full),
        ],
        out_specs=[
            pl.BlockSpec((_NG, _NCLS), full),
            pl.BlockSpec((_NG, _H), full),
        ],
        out_shape=[
            jax.ShapeDtypeStruct((_NG, _NCLS), jnp.float32),
            jax.ShapeDtypeStruct((_NG, _H), jnp.float32),
        ],
        scratch_shapes=[
            pltpu.VMEM((_NG, _H), jnp.float32),
            pltpu.VMEM((_NG, 8), jnp.float32),
        ],
    )(q, hs2, degp, batch2, b2r, bngr, bnbr, bn1gr, bn1br,
      l1W, l1br, l2W, l2br, l3W, l3br)


def kernel(x, edge_index, batch, W1, b1, W2, b2, bng, bnb, bn1g, bn1b,
           l1W, l1b, l2W, l2b, l3W, l3b):
    src = edge_index[0].astype(jnp.int32)
    dst = edge_index[1].astype(jnp.int32)
    dst3 = dst.reshape(_NW, _CHD, _KD)
    packed3 = (src | (dst << 16)).reshape(_NW, _CH, _K)
    batch2 = batch.astype(jnp.int32).reshape(_N, 1)

    onesrow = jnp.concatenate(
        [jnp.ones((_KD, 1), jnp.float32), jnp.zeros((_KD, 15), jnp.float32)],
        axis=1)
    zrows = jnp.zeros((_NS, _RPT, 16), jnp.float32)
    zblk = jnp.zeros((_NS, _RPT, _H), jnp.float32)

    degp = _deg_partial(dst3, onesrow, zrows).reshape(_NC, _N, 16)
    h0 = _matmul(x, W1)
    hs1 = _scale(h0, degp)
    p = _prop_partial(hs1, packed3, zblk).reshape(_NC, _N, _H)
    hs2 = _layer2(p, hs1, degp, b1.reshape(1, _H), W2)
    q = _prop_partial(hs2, packed3, zblk).reshape(_NC, _N, _H)
    out, xp = _final(
        q, hs2, degp, batch2,
        b2.reshape(1, _H), bng.reshape(1, _H), bnb.reshape(1, _H),
        bn1g.reshape(1, _H), bn1b.reshape(1, _H),
        l1W, l1b.reshape(1, _DENSE), l2W, l2b.reshape(1, _DENSE),
        l3W, l3b.reshape(1, _NCLS))
    return (out, xp)


# speedup vs baseline: 2.9339x; 1.0676x over previous
import functools

import numpy as np
import jax
import jax.numpy as jnp
from jax import lax
from jax.experimental import pallas as pl
from jax.experimental.pallas import tpu as pltpu
from jax.experimental.pallas import tpu_sc as plsc

_N = 10000
_E = 320000
_D = 128
_H = 128
_DENSE = 64
_NCLS = 2
_NG = 128
_EPS = 1e-5

_NC = 2
_NS = 16
_NW = _NC * _NS
_K = 80
_CH = _E // (_NW * _K)
_CHD = 80
_KD = 125
_RPT = _N // _NS

_BM = 1000
_G = _N // _BM

_DRN = 624
_DTL = _N - _NS * _DRN

_BNS = np.float32(1.0 / np.sqrt(1.0 + _EPS))


def _sc_mesh():
    return plsc.VectorSubcoreMesh(core_axis_name="c", subcore_axis_name="s")


def _deg_partial(dst3, onesrow, zrows):
    @functools.partial(
        pl.kernel,
        out_type=jax.ShapeDtypeStruct((_NC, _N, 16), jnp.float32),
        mesh=_sc_mesh(),
        scratch_types=[
            pltpu.VMEM((_CHD, _KD), jnp.int32),
            pltpu.VMEM((_KD, 16), jnp.float32),
            pltpu.VMEM_SHARED((_N, 16), jnp.float32),
            pltpu.SemaphoreType.DMA,
        ],
    )
    def k(dst_hbm, ones_hbm, z_hbm, out_hbm, dstv, ones, acc, sem):
        c = lax.axis_index("c")
        s = lax.axis_index("s")
        tid = c * _NS + s
        pltpu.sync_copy(dst_hbm.at[tid], dstv)
        pltpu.sync_copy(ones_hbm, ones)
        base = s * _RPT
        pltpu.sync_copy(z_hbm.at[s], acc.at[pl.ds(base, _RPT)])
        plsc.subcore_barrier()

        @pl.loop(0, _CHD, step=16)
        def _(ci):
            for j in range(16):
                pltpu.async_copy(ones, acc.at[dstv.at[ci + j]], sem, add=True)
            for j in range(16):
                pltpu.make_async_copy(ones, acc.at[dstv.at[ci + j]], sem).wait()

        plsc.subcore_barrier()
        dbase = s * _DRN
        pltpu.sync_copy(acc.at[pl.ds(dbase, _DRN)],
                        out_hbm.at[c, pl.ds(dbase, _DRN)])

        @pl.when(s == _NS - 1)
        def _():
            pltpu.sync_copy(acc.at[pl.ds(_NS * _DRN, _DTL)],
                            out_hbm.at[c, pl.ds(_NS * _DRN, _DTL)])

    return k(dst3, onesrow, zrows)


def _prop_partial(hs, packed3, zblk):
    def unpack(pk_row, srcb, dstb):
        for j in range(_K // 16):
            pk = pk_row[pl.ds(j * 16, 16)]
            srcb[pl.ds(j * 16, 16)] = lax.bitwise_and(pk, 0xFFFF)
            dstb[pl.ds(j * 16, 16)] = lax.shift_right_logical(pk, 16)

    @functools.partial(
        pl.kernel,
        out_type=jax.ShapeDtypeStruct((_NC, _N, _H), jnp.float32),
        mesh=_sc_mesh(),
        scratch_types=[
            pltpu.VMEM((_CH, _K), jnp.int32),
            pltpu.VMEM((_K,), jnp.int32),
            pltpu.VMEM((_K,), jnp.int32),
            pltpu.VMEM((_K,), jnp.int32),
            pltpu.VMEM((_K,), jnp.int32),
            pltpu.VMEM((_K, _H), jnp.float32),
            pltpu.VMEM((_K, _H), jnp.float32),
            pltpu.VMEM_SHARED((_N, _H), jnp.float32),
            pltpu.SemaphoreType.DMA,
            pltpu.SemaphoreType.DMA,
        ],
    )
    def k(hs_hbm, pk_hbm, z_hbm, out_hbm,
          pkv, src_a, dst_a, src_b, dst_b, rows_a, rows_b, acc, sem_a, sem_b):
        c = lax.axis_index("c")
        s = lax.axis_index("s")
        tid = c * _NS + s
        pltpu.sync_copy(pk_hbm.at[tid], pkv)
        base = s * _RPT
        pltpu.sync_copy(z_hbm.at[s], acc.at[pl.ds(base, _RPT)])
        plsc.subcore_barrier()

        @pl.loop(0, _CH - 1, step=2)
        def _(ci):
            @pl.when(ci >= 2)
            def _():
                pltpu.make_async_copy(rows_a, acc.at[dst_a], sem_a).wait()

            unpack(pkv.at[ci], src_a, dst_a)
            pltpu.sync_copy(hs_hbm.at[src_a], rows_a)
            pltpu.async_copy(rows_a, acc.at[dst_a], sem_a, add=True)

            @pl.when(ci >= 2)
            def _():
                pltpu.make_async_copy(rows_b, acc.at[dst_b], sem_b).wait()

            unpack(pkv.at[ci + 1], src_b, dst_b)
            pltpu.sync_copy(hs_hbm.at[src_b], rows_b)
            pltpu.async_copy(rows_b, acc.at[dst_b], sem_b, add=True)

        pltpu.make_async_copy(rows_a, acc.at[dst_a], sem_a).wait()
        unpack(pkv.at[_CH - 1], src_a, dst_a)
        pltpu.sync_copy(hs_hbm.at[src_a], rows_a)
        pltpu.async_copy(rows_a, acc.at[dst_a], sem_a, add=True)

        pltpu.make_async_copy(rows_a, acc.at[dst_a], sem_a).wait()
        pltpu.make_async_copy(rows_b, acc.at[dst_b], sem_b).wait()
        plsc.subcore_barrier()
        dbase = s * _DRN
        pltpu.sync_copy(acc.at[pl.ds(dbase, _DRN)],
                        out_hbm.at[c, pl.ds(dbase, _DRN)])

        @pl.when(s == _NS - 1)
        def _():
            pltpu.sync_copy(acc.at[pl.ds(_NS * _DRN, _DTL)],
                            out_hbm.at[c, pl.ds(_NS * _DRN, _DTL)])

    return k(hs, packed3, zblk)


def _matmul(x, W):
    def body(x_ref, w_ref, o_ref):
        o_ref[...] = jnp.dot(x_ref[...], w_ref[...],
                             preferred_element_type=jnp.float32)

    return pl.pallas_call(
        body,
        grid=(_G,),
        in_specs=[
            pl.BlockSpec((_BM, _D), lambda i: (i, 0)),
            pl.BlockSpec((_D, _H), lambda i: (0, 0)),
        ],
        out_specs=pl.BlockSpec((_BM, _H), lambda i: (i, 0)),
        out_shape=jax.ShapeDtypeStruct((_N, _H), jnp.float32),
    )(x, W)


def _dinv_of(degp_ref):
    deg = degp_ref[0, :, 0:1] + degp_ref[1, :, 0:1] + 1.0
    return lax.rsqrt(deg)


def _scale(h, degp):
    def body(h_ref, degp_ref, o_ref):
        o_ref[...] = h_ref[...] * _dinv_of(degp_ref)

    return pl.pallas_call(
        body,
        grid=(_G,),
        in_specs=[
            pl.BlockSpec((_BM, _H), lambda i: (i, 0)),
            pl.BlockSpec((_NC, _BM, 16), lambda i: (0, i, 0)),
        ],
        out_specs=pl.BlockSpec((_BM, _H), lambda i: (i, 0)),
        out_shape=jax.ShapeDtypeStruct((_N, _H), jnp.float32),
    )(h, degp)


def _layer2(p, hs1, degp, b1r, W2):
    def body(p_ref, hs1_ref, degp_ref, b1_ref, w2_ref, o_ref):
        dinv = _dinv_of(degp_ref)
        h1 = jnp.maximum(
            (p_ref[0] + p_ref[1] + hs1_ref[...]) * dinv + b1_ref[...], 0.0)
        o_ref[...] = jnp.dot(h1, w2_ref[...],
                             preferred_element_type=jnp.float32) * dinv

    return pl.pallas_call(
        body,
        grid=(_G,),
        in_specs=[
            pl.BlockSpec((_NC, _BM, _H), lambda i: (0, i, 0)),
            pl.BlockSpec((_BM, _H), lambda i: (i, 0)),
            pl.BlockSpec((_NC, _BM, 16), lambda i: (0, i, 0)),
            pl.BlockSpec((1, _H), lambda i: (0, 0)),
            pl.BlockSpec((_H, _H), lambda i: (0, 0)),
        ],
        out_specs=pl.BlockSpec((_BM, _H), lambda i: (i, 0)),
        out_shape=jax.ShapeDtypeStruct((_N, _H), jnp.float32),
    )(p, hs1, degp, b1r, W2)


def _final(q, hs2, degp, batch2, b2r, bngr, bnbr, bn1gr, bn1br,
           l1W, l1br, l2W, l2br, l3W, l3br):
    def body(q_ref, hs2_ref, degp_ref, batch_ref, b2_ref, bng_ref, bnb_ref,
             bn1g_ref, bn1b_ref, l1w_ref, l1b_ref, l2w_ref, l2b_ref,
             l3w_ref, l3b_ref, out_ref, xp_ref, sums_ref, cnts_ref):
        i = pl.program_id(0)

        @pl.when(i == 0)
        def _():
            sums_ref[...] = jnp.zeros_like(sums_ref)
            cnts_ref[...] = jnp.zeros_like(cnts_ref)

        dinv = _dinv_of(degp_ref)
        pre = (q_ref[0] + q_ref[1] + hs2_ref[...]) * dinv + b2_ref[...]
        h2 = jnp.maximum(pre * (bng_ref[...] * _BNS) + bnb_ref[...], 0.0)
        oh = (batch_ref[...] ==
              lax.broadcasted_iota(jnp.int32, (1, _NG), 1)).astype(jnp.float32)
        sums_ref[...] += lax.dot_general(
            oh, h2, (((0,), (0,)), ((), ())),
            preferred_element_type=jnp.float32)
        cnts_ref[...] += lax.dot_general(
            oh, jnp.ones((_BM, 8), jnp.float32), (((0,), (0,)), ((), ())),
            preferred_element_type=jnp.float32)

        @pl.when(i == _G - 1)
        def _():
            xp = sums_ref[...] / jnp.maximum(cnts_ref[:, 0:1], 1.0)
            xb = xp * (bn1g_ref[...] * _BNS) + bn1b_ref[...]
            a = jnp.maximum(
                jnp.dot(xb, l1w_ref[...],
                        preferred_element_type=jnp.float32) + l1b_ref[...], 0.0)
            a = jnp.maximum(
                jnp.dot(a, l2w_ref[...],
                        preferred_element_type=jnp.float32) + l2b_ref[...], 0.0)
            a = jnp.maximum(
                jnp.dot(a, l3w_ref[...],
                        preferred_element_type=jnp.float32) + l3b_ref[...], 0.0)
            m = jnp.max(a, axis=1, keepdims=True)
            e = jnp.exp(a - m)
            out_ref[...] = e / jnp.sum(e, axis=1, keepdims=True)
            xp_ref[...] = xp

    full = lambda i: (0, 0)
    return pl.pallas_call(
        body,
        grid=(_G,),
        in_specs=[
            pl.BlockSpec((_NC, _BM, _H), lambda i: (0, i, 0)),
            pl.BlockSpec((_BM, _H), lambda i: (i, 0)),
            pl.BlockSpec((_NC, _BM, 16), lambda i: (0, i, 0)),
            pl.BlockSpec((_BM, 1), lambda i: (i, 0)),
            pl.BlockSpec((1, _H), full),
            pl.BlockSpec((1, _H), full),
            pl.BlockSpec((1, _H), full),
            pl.BlockSpec((1, _H), full),
            pl.BlockSpec((1, _H), full),
            pl.BlockSpec((_H, _DENSE), full),
            pl.BlockSpec((1, _DENSE), full),
            pl.BlockSpec((_DENSE, _DENSE), full),
            pl.BlockSpec((1, _DENSE), full),
            pl.BlockSpec((_DENSE, _NCLS), full),
            pl.BlockSpec((1, _NCLS), full),
        ],
        out_specs=[
            pl.BlockSpec((_NG, _NCLS), full),
            pl.BlockSpec((_NG, _H), full),
        ],
        out_shape=[
            jax.ShapeDtypeStruct((_NG, _NCLS), jnp.float32),
            jax.ShapeDtypeStruct((_NG, _H), jnp.float32),
        ],
        scratch_shapes=[
            pltpu.VMEM((_NG, _H), jnp.float32),
            pltpu.VMEM((_NG, 8), jnp.float32),
        ],
    )(q, hs2, degp, batch2, b2r, bngr, bnbr, bn1gr, bn1br,
      l1W, l1br, l2W, l2br, l3W, l3br)


def kernel(x, edge_index, batch, W1, b1, W2, b2, bng, bnb, bn1g, bn1b,
           l1W, l1b, l2W, l2b, l3W, l3b):
    src = edge_index[0].astype(jnp.int32)
    dst = edge_index[1].astype(jnp.int32)
    dst3 = dst.reshape(_NW, _CHD, _KD)
    packed3 = (src | (dst << 16)).reshape(_NW, _CH, _K)
    batch2 = batch.astype(jnp.int32).reshape(_N, 1)

    onesrow = jnp.concatenate(
        [jnp.ones((_KD, 1), jnp.float32), jnp.zeros((_KD, 15), jnp.float32)],
        axis=1)
    zrows = jnp.zeros((_NS, _RPT, 16), jnp.float32)
    zblk = jnp.zeros((_NS, _RPT, _H), jnp.float32)

    degp = _deg_partial(dst3, onesrow, zrows)
    h0 = _matmul(x, W1)
    hs1 = _scale(h0, degp)
    p = _prop_partial(hs1, packed3, zblk)
    hs2 = _layer2(p, hs1, degp, b1.reshape(1, _H), W2)
    q = _prop_partial(hs2, packed3, zblk)
    out, xp = _final(
        q, hs2, degp, batch2,
        b2.reshape(1, _H), bng.reshape(1, _H), bnb.reshape(1, _H),
        bn1g.reshape(1, _H), bn1b.reshape(1, _H),
        l1W, l1b.reshape(1, _DENSE), l2W, l2b.reshape(1, _DENSE),
        l3W, l3b.reshape(1, _NCLS))
    return (out, xp)


# speedup vs baseline: 2.9356x; 1.0006x over previous
import functools

import numpy as np
import jax
import jax.numpy as jnp
from jax import lax
from jax.experimental import pallas as pl
from jax.experimental.pallas import tpu as pltpu
from jax.experimental.pallas import tpu_sc as plsc

_N = 10000
_E = 320000
_D = 128
_H = 128
_DENSE = 64
_NCLS = 2
_NG = 128
_EPS = 1e-5

_NC = 2
_NS = 16
_NW = _NC * _NS
_K = 80
_CH = _E // (_NW * _K)
_CHD = 80
_KD = 125
_RPT = _N // _NS

_BM = 1000
_G = _N // _BM

_DRN = 624
_DTL = _N - _NS * _DRN

_BNS = np.float32(1.0 / np.sqrt(1.0 + _EPS))


def _sc_mesh():
    return plsc.VectorSubcoreMesh(core_axis_name="c", subcore_axis_name="s")


def _deg_partial(dst3, onesrow, zrows):
    @functools.partial(
        pl.kernel,
        out_type=jax.ShapeDtypeStruct((_NC, _N, 16), jnp.float32),
        mesh=_sc_mesh(),
        scratch_types=[
            pltpu.VMEM((_CHD, _KD), jnp.int32),
            pltpu.VMEM((_KD, 16), jnp.float32),
            pltpu.VMEM_SHARED((_N, 16), jnp.float32),
            pltpu.SemaphoreType.DMA,
        ],
    )
    def k(dst_hbm, ones_hbm, z_hbm, out_hbm, dstv, ones, acc, sem):
        c = lax.axis_index("c")
        s = lax.axis_index("s")
        tid = c * _NS + s
        pltpu.sync_copy(dst_hbm.at[tid], dstv)
        pltpu.sync_copy(ones_hbm, ones)
        base = s * _RPT
        pltpu.sync_copy(z_hbm.at[s], acc.at[pl.ds(base, _RPT)])
        plsc.subcore_barrier()

        @pl.loop(0, _CHD, step=16)
        def _(ci):
            for j in range(16):
                pltpu.async_copy(ones, acc.at[dstv.at[ci + j]], sem, add=True)
            for j in range(16):
                pltpu.make_async_copy(ones, acc.at[dstv.at[ci + j]], sem).wait()

        plsc.subcore_barrier()
        dbase = s * _DRN
        pltpu.sync_copy(acc.at[pl.ds(dbase, _DRN)],
                        out_hbm.at[c, pl.ds(dbase, _DRN)])

        @pl.when(s == _NS - 1)
        def _():
            pltpu.sync_copy(acc.at[pl.ds(_NS * _DRN, _DTL)],
                            out_hbm.at[c, pl.ds(_NS * _DRN, _DTL)])

    return k(dst3, onesrow, zrows)


def _prop_partial(hs, packed3, zblk):
    def unpack(pk_row, srcb, dstb):
        for j in range(_K // 16):
            pk = pk_row[pl.ds(j * 16, 16)]
            srcb[pl.ds(j * 16, 16)] = lax.bitwise_and(pk, 0xFFFF)
            dstb[pl.ds(j * 16, 16)] = lax.shift_right_logical(pk, 16)

    @functools.partial(
        pl.kernel,
        out_type=jax.ShapeDtypeStruct((_NC, _N, _H), jnp.float32),
        mesh=_sc_mesh(),
        scratch_types=[
            pltpu.VMEM((_CH, _K), jnp.int32),
            pltpu.VMEM((_K,), jnp.int32),
            pltpu.VMEM((_K,), jnp.int32),
            pltpu.VMEM((_K,), jnp.int32),
            pltpu.VMEM((_K,), jnp.int32),
            pltpu.VMEM((_K, _H), jnp.float32),
            pltpu.VMEM((_K, _H), jnp.float32),
            pltpu.VMEM_SHARED((_N, _H), jnp.float32),
            pltpu.SemaphoreType.DMA,
            pltpu.SemaphoreType.DMA,
        ],
    )
    def k(hs_hbm, pk_hbm, z_hbm, out_hbm,
          pkv, src_a, dst_a, src_b, dst_b, rows_a, rows_b, acc, sem_a, sem_b):
        c = lax.axis_index("c")
        s = lax.axis_index("s")
        tid = c * _NS + s
        pltpu.sync_copy(pk_hbm.at[tid], pkv)
        base = s * _RPT
        pltpu.sync_copy(z_hbm.at[s], acc.at[pl.ds(base, _RPT)])
        plsc.subcore_barrier()

        @pl.loop(0, _CH - 1, step=2)
        def _(ci):
            @pl.when(ci >= 2)
            def _():
                pltpu.make_async_copy(rows_a, acc.at[dst_a], sem_a).wait()

            unpack(pkv.at[ci], src_a, dst_a)
            pltpu.sync_copy(hs_hbm.at[src_a], rows_a)
            pltpu.async_copy(rows_a, acc.at[dst_a], sem_a, add=True)

            @pl.when(ci >= 2)
            def _():
                pltpu.make_async_copy(rows_b, acc.at[dst_b], sem_b).wait()

            unpack(pkv.at[ci + 1], src_b, dst_b)
            pltpu.sync_copy(hs_hbm.at[src_b], rows_b)
            pltpu.async_copy(rows_b, acc.at[dst_b], sem_b, add=True)

        pltpu.make_async_copy(rows_a, acc.at[dst_a], sem_a).wait()
        unpack(pkv.at[_CH - 1], src_a, dst_a)
        pltpu.sync_copy(hs_hbm.at[src_a], rows_a)
        pltpu.async_copy(rows_a, acc.at[dst_a], sem_a, add=True)

        pltpu.make_async_copy(rows_a, acc.at[dst_a], sem_a).wait()
        pltpu.make_async_copy(rows_b, acc.at[dst_b], sem_b).wait()
        plsc.subcore_barrier()
        dbase = s * _DRN
        pltpu.sync_copy(acc.at[pl.ds(dbase, _DRN)],
                        out_hbm.at[c, pl.ds(dbase, _DRN)])

        @pl.when(s == _NS - 1)
        def _():
            pltpu.sync_copy(acc.at[pl.ds(_NS * _DRN, _DTL)],
                            out_hbm.at[c, pl.ds(_NS * _DRN, _DTL)])

    return k(hs, packed3, zblk)


def _dinv_of(degp_ref):
    deg = degp_ref[0, :, 0:1] + degp_ref[1, :, 0:1] + 1.0
    return lax.rsqrt(deg)


def _mm_scale(x, W, degp):
    def body(x_ref, w_ref, degp_ref, o_ref):
        o_ref[...] = jnp.dot(x_ref[...], w_ref[...],
                             preferred_element_type=jnp.float32) * _dinv_of(degp_ref)

    return pl.pallas_call(
        body,
        grid=(_G,),
        in_specs=[
            pl.BlockSpec((_BM, _D), lambda i: (i, 0)),
            pl.BlockSpec((_D, _H), lambda i: (0, 0)),
            pl.BlockSpec((_NC, _BM, 16), lambda i: (0, i, 0)),
        ],
        out_specs=pl.BlockSpec((_BM, _H), lambda i: (i, 0)),
        out_shape=jax.ShapeDtypeStruct((_N, _H), jnp.float32),
    )(x, W, degp)


def _layer2(p, hs1, degp, b1r, W2):
    def body(p_ref, hs1_ref, degp_ref, b1_ref, w2_ref, o_ref):
        dinv = _dinv_of(degp_ref)
        h1 = jnp.maximum(
            (p_ref[0] + p_ref[1] + hs1_ref[...]) * dinv + b1_ref[...], 0.0)
        o_ref[...] = jnp.dot(h1, w2_ref[...],
                             preferred_element_type=jnp.float32) * dinv

    return pl.pallas_call(
        body,
        grid=(_G,),
        in_specs=[
            pl.BlockSpec((_NC, _BM, _H), lambda i: (0, i, 0)),
            pl.BlockSpec((_BM, _H), lambda i: (i, 0)),
            pl.BlockSpec((_NC, _BM, 16), lambda i: (0, i, 0)),
            pl.BlockSpec((1, _H), lambda i: (0, 0)),
            pl.BlockSpec((_H, _H), lambda i: (0, 0)),
        ],
        out_specs=pl.BlockSpec((_BM, _H), lambda i: (i, 0)),
        out_shape=jax.ShapeDtypeStruct((_N, _H), jnp.float32),
    )(p, hs1, degp, b1r, W2)


def _final(q, hs2, degp, batch2, b2r, bngr, bnbr, bn1gr, bn1br,
           l1W, l1br, l2W, l2br, l3W, l3br):
    def body(q_ref, hs2_ref, degp_ref, batch_ref, b2_ref, bng_ref, bnb_ref,
             bn1g_ref, bn1b_ref, l1w_ref, l1b_ref, l2w_ref, l2b_ref,
             l3w_ref, l3b_ref, out_ref, xp_ref, sums_ref, cnts_ref):
        i = pl.program_id(0)

        @pl.when(i == 0)
        def _():
            sums_ref[...] = jnp.zeros_like(sums_ref)
            cnts_ref[...] = jnp.zeros_like(cnts_ref)

        dinv = _dinv_of(degp_ref)
        pre = (q_ref[0] + q_ref[1] + hs2_ref[...]) * dinv + b2_ref[...]
        h2 = jnp.maximum(pre * (bng_ref[...] * _BNS) + bnb_ref[...], 0.0)
        oh = (batch_ref[...] ==
              lax.broadcasted_iota(jnp.int32, (1, _NG), 1)).astype(jnp.float32)
        sums_ref[...] += lax.dot_general(
            oh, h2, (((0,), (0,)), ((), ())),
            preferred_element_type=jnp.float32)
        cnts_ref[...] += lax.dot_general(
            oh, jnp.ones((_BM, 8), jnp.float32), (((0,), (0,)), ((), ())),
            preferred_element_type=jnp.float32)

        @pl.when(i == _G - 1)
        def _():
            xp = sums_ref[...] / jnp.maximum(cnts_ref[:, 0:1], 1.0)
            xb = xp * (bn1g_ref[...] * _BNS) + bn1b_ref[...]
            a = jnp.maximum(
                jnp.dot(xb, l1w_ref[...],
                        preferred_element_type=jnp.float32) + l1b_ref[...], 0.0)
            a = jnp.maximum(
                jnp.dot(a, l2w_ref[...],
                        preferred_element_type=jnp.float32) + l2b_ref[...], 0.0)
            a = jnp.maximum(
                jnp.dot(a, l3w_ref[...],
                        preferred_element_type=jnp.float32) + l3b_ref[...], 0.0)
            m = jnp.max(a, axis=1, keepdims=True)
            e = jnp.exp(a - m)
            out_ref[...] = e / jnp.sum(e, axis=1, keepdims=True)
            xp_ref[...] = xp

    full = lambda i: (0, 0)
    return pl.pallas_call(
        body,
        grid=(_G,),
        in_specs=[
            pl.BlockSpec((_NC, _BM, _H), lambda i: (0, i, 0)),
            pl.BlockSpec((_BM, _H), lambda i: (i, 0)),
            pl.BlockSpec((_NC, _BM, 16), lambda i: (0, i, 0)),
            pl.BlockSpec((_BM, 1), lambda i: (i, 0)),
            pl.BlockSpec((1, _H), full),
            pl.BlockSpec((1, _H), full),
            pl.BlockSpec((1, _H), full),
            pl.BlockSpec((1, _H), full),
            pl.BlockSpec((1, _H), full),
            pl.BlockSpec((_H, _DENSE), full),
            pl.BlockSpec((1, _DENSE), full),
            pl.BlockSpec((_DENSE, _DENSE), full),
            pl.BlockSpec((1, _DENSE), full),
            pl.BlockSpec((_DENSE, _NCLS), full),
            pl.BlockSpec((1, _NCLS), full),
        ],
        out_specs=[
            pl.BlockSpec((_NG, _NCLS), full),
            pl.BlockSpec((_NG, _H), full),
        ],
        out_shape=[
            jax.ShapeDtypeStruct((_NG, _NCLS), jnp.float32),
            jax.ShapeDtypeStruct((_NG, _H), jnp.float32),
        ],
        scratch_shapes=[
            pltpu.VMEM((_NG, _H), jnp.float32),
            pltpu.VMEM((_NG, 8), jnp.float32),
        ],
    )(q, hs2, degp, batch2, b2r, bngr, bnbr, bn1gr, bn1br,
      l1W, l1br, l2W, l2br, l3W, l3br)


def kernel(x, edge_index, batch, W1, b1, W2, b2, bng, bnb, bn1g, bn1b,
           l1W, l1b, l2W, l2b, l3W, l3b):
    src = edge_index[0].astype(jnp.int32)
    dst = edge_index[1].astype(jnp.int32)
    dst3 = dst.reshape(_NW, _CHD, _KD)
    packed3 = (src | (dst << 16)).reshape(_NW, _CH, _K)
    batch2 = batch.astype(jnp.int32).reshape(_N, 1)

    onesrow = jnp.concatenate(
        [jnp.ones((_KD, 1), jnp.float32), jnp.zeros((_KD, 15), jnp.float32)],
        axis=1)
    zrows = jnp.zeros((_NS, _RPT, 16), jnp.float32)
    zblk = jnp.zeros((_NS, _RPT, _H), jnp.float32)

    degp = _deg_partial(dst3, onesrow, zrows)
    hs1 = _mm_scale(x, W1, degp)
    p = _prop_partial(hs1, packed3, zblk)
    hs2 = _layer2(p, hs1, degp, b1.reshape(1, _H), W2)
    q = _prop_partial(hs2, packed3, zblk)
    out, xp = _final(
        q, hs2, degp, batch2,
        b2.reshape(1, _H), bng.reshape(1, _H), bnb.reshape(1, _H),
        bn1g.reshape(1, _H), bn1b.reshape(1, _H),
        l1W, l1b.reshape(1, _DENSE), l2W, l2b.reshape(1, _DENSE),
        l3W, l3b.reshape(1, _NCLS))
    return (out, xp)


# speedup vs baseline: 2.9847x; 1.0167x over previous
import functools

import numpy as np
import jax
import jax.numpy as jnp
from jax import lax
from jax.experimental import pallas as pl
from jax.experimental.pallas import tpu as pltpu
from jax.experimental.pallas import tpu_sc as plsc

_N = 10000
_E = 320000
_D = 128
_H = 128
_DENSE = 64
_NCLS = 2
_NG = 128
_EPS = 1e-5

_NC = 2
_NS = 16
_NW = _NC * _NS
_K = 80
_CH = _E // (_NW * _K)
_CHD = 80
_KD = 125
_RPT = _N // _NS

_BM = 1000
_G = _N // _BM

_DRN = 624
_DTL = _N - _NS * _DRN

_BNS = np.float32(1.0 / np.sqrt(1.0 + _EPS))


def _sc_mesh():
    return plsc.VectorSubcoreMesh(core_axis_name="c", subcore_axis_name="s")


def _deg_partial(dst3, onesrow, zrows):
    @functools.partial(
        pl.kernel,
        out_type=jax.ShapeDtypeStruct((_NC, _N, 16), jnp.float32),
        mesh=_sc_mesh(),
        scratch_types=[
            pltpu.VMEM((_CHD, _KD), jnp.int32),
            pltpu.VMEM((_KD, 16), jnp.float32),
            pltpu.VMEM_SHARED((_N, 16), jnp.float32),
            pltpu.SemaphoreType.DMA,
        ],
    )
    def k(dst_hbm, ones_hbm, z_hbm, out_hbm, dstv, ones, acc, sem):
        c = lax.axis_index("c")
        s = lax.axis_index("s")
        tid = c * _NS + s
        pltpu.sync_copy(dst_hbm.at[tid], dstv)
        pltpu.sync_copy(ones_hbm, ones)
        base = s * _RPT
        pltpu.sync_copy(z_hbm.at[s], acc.at[pl.ds(base, _RPT)])
        plsc.subcore_barrier()

        @pl.loop(0, _CHD, step=16)
        def _(ci):
            for j in range(16):
                pltpu.async_copy(ones, acc.at[dstv.at[ci + j]], sem, add=True)
            for j in range(16):
                pltpu.make_async_copy(ones, acc.at[dstv.at[ci + j]], sem).wait()

        plsc.subcore_barrier()
        dbase = s * _DRN
        pltpu.sync_copy(acc.at[pl.ds(dbase, _DRN)],
                        out_hbm.at[c, pl.ds(dbase, _DRN)])

        @pl.when(s == _NS - 1)
        def _():
            pltpu.sync_copy(acc.at[pl.ds(_NS * _DRN, _DTL)],
                            out_hbm.at[c, pl.ds(_NS * _DRN, _DTL)])

    return k(dst3, onesrow, zrows)


def _prop_partial(hs, packed3, zblk):
    def unpack(pk_row, srcb, dstb):
        for j in range(_K // 16):
            pk = pk_row[pl.ds(j * 16, 16)]
            srcb[pl.ds(j * 16, 16)] = lax.bitwise_and(pk, 0xFFFF)
            dstb[pl.ds(j * 16, 16)] = lax.shift_right_logical(pk, 16)

    @functools.partial(
        pl.kernel,
        out_type=jax.ShapeDtypeStruct((_NC, _N, _H), jnp.float32),
        mesh=_sc_mesh(),
        scratch_types=[
            pltpu.VMEM((_CH, _K), jnp.int32),
            pltpu.VMEM((_K,), jnp.int32),
            pltpu.VMEM((_K,), jnp.int32),
            pltpu.VMEM((_K,), jnp.int32),
            pltpu.VMEM((_K,), jnp.int32),
            pltpu.VMEM((_K, _H), jnp.float32),
            pltpu.VMEM((_K, _H), jnp.float32),
            pltpu.VMEM_SHARED((_N, _H), jnp.float32),
            pltpu.SemaphoreType.DMA,
            pltpu.SemaphoreType.DMA,
            pltpu.SemaphoreType.DMA,
            pltpu.SemaphoreType.DMA,
        ],
    )
    def k(hs_hbm, pk_hbm, z_hbm, out_hbm,
          pkv, src_a, dst_a, src_b, dst_b, rows_a, rows_b, acc,
          sem_a, sem_b, sem_g, sem_g2):
        c = lax.axis_index("c")
        s = lax.axis_index("s")
        tid = c * _NS + s
        pltpu.sync_copy(pk_hbm.at[tid], pkv)
        base = s * _RPT
        pltpu.sync_copy(z_hbm.at[s], acc.at[pl.ds(base, _RPT)])
        plsc.subcore_barrier()

        @pl.loop(0, _CH - 1, step=2)
        def _(ci):
            @pl.when(ci >= 2)
            def _():
                pltpu.make_async_copy(rows_a, acc.at[dst_a], sem_a).wait()

            unpack(pkv.at[ci], src_a, dst_a)
            ga = pltpu.async_copy(hs_hbm.at[src_a], rows_a, sem_g)

            @pl.when(ci >= 2)
            def _():
                pltpu.make_async_copy(rows_b, acc.at[dst_b], sem_b).wait()

            unpack(pkv.at[ci + 1], src_b, dst_b)
            gb = pltpu.async_copy(hs_hbm.at[src_b], rows_b, sem_g2)
            ga.wait()
            pltpu.async_copy(rows_a, acc.at[dst_a], sem_a, add=True)
            gb.wait()
            pltpu.async_copy(rows_b, acc.at[dst_b], sem_b, add=True)

        pltpu.make_async_copy(rows_a, acc.at[dst_a], sem_a).wait()
        unpack(pkv.at[_CH - 1], src_a, dst_a)
        pltpu.sync_copy(hs_hbm.at[src_a], rows_a)
        pltpu.async_copy(rows_a, acc.at[dst_a], sem_a, add=True)

        pltpu.make_async_copy(rows_a, acc.at[dst_a], sem_a).wait()
        pltpu.make_async_copy(rows_b, acc.at[dst_b], sem_b).wait()
        plsc.subcore_barrier()
        dbase = s * _DRN
        pltpu.sync_copy(acc.at[pl.ds(dbase, _DRN)],
                        out_hbm.at[c, pl.ds(dbase, _DRN)])

        @pl.when(s == _NS - 1)
        def _():
            pltpu.sync_copy(acc.at[pl.ds(_NS * _DRN, _DTL)],
                            out_hbm.at[c, pl.ds(_NS * _DRN, _DTL)])

    return k(hs, packed3, zblk)


def _dinv_of(degp_ref):
    deg = degp_ref[0, :, 0:1] + degp_ref[1, :, 0:1] + 1.0
    return lax.rsqrt(deg)


def _mm_scale(x, W, degp):
    def body(x_ref, w_ref, degp_ref, o_ref):
        o_ref[...] = jnp.dot(x_ref[...], w_ref[...],
                             preferred_element_type=jnp.float32) * _dinv_of(degp_ref)

    return pl.pallas_call(
        body,
        grid=(_G,),
        in_specs=[
            pl.BlockSpec((_BM, _D), lambda i: (i, 0)),
            pl.BlockSpec((_D, _H), lambda i: (0, 0)),
            pl.BlockSpec((_NC, _BM, 16), lambda i: (0, i, 0)),
        ],
        out_specs=pl.BlockSpec((_BM, _H), lambda i: (i, 0)),
        out_shape=jax.ShapeDtypeStruct((_N, _H), jnp.float32),
    )(x, W, degp)


def _layer2(p, hs1, degp, b1r, W2):
    def body(p_ref, hs1_ref, degp_ref, b1_ref, w2_ref, o_ref):
        dinv = _dinv_of(degp_ref)
        h1 = jnp.maximum(
            (p_ref[0] + p_ref[1] + hs1_ref[...]) * dinv + b1_ref[...], 0.0)
        o_ref[...] = jnp.dot(h1, w2_ref[...],
                             preferred_element_type=jnp.float32) * dinv

    return pl.pallas_call(
        body,
        grid=(_G,),
        in_specs=[
            pl.BlockSpec((_NC, _BM, _H), lambda i: (0, i, 0)),
            pl.BlockSpec((_BM, _H), lambda i: (i, 0)),
            pl.BlockSpec((_NC, _BM, 16), lambda i: (0, i, 0)),
            pl.BlockSpec((1, _H), lambda i: (0, 0)),
            pl.BlockSpec((_H, _H), lambda i: (0, 0)),
        ],
        out_specs=pl.BlockSpec((_BM, _H), lambda i: (i, 0)),
        out_shape=jax.ShapeDtypeStruct((_N, _H), jnp.float32),
    )(p, hs1, degp, b1r, W2)


def _final(q, hs2, degp, batch2, b2r, bngr, bnbr, bn1gr, bn1br,
           l1W, l1br, l2W, l2br, l3W, l3br):
    def body(q_ref, hs2_ref, degp_ref, batch_ref, b2_ref, bng_ref, bnb_ref,
             bn1g_ref, bn1b_ref, l1w_ref, l1b_ref, l2w_ref, l2b_ref,
             l3w_ref, l3b_ref, out_ref, xp_ref, sums_ref, cnts_ref):
        i = pl.program_id(0)

        @pl.when(i == 0)
        def _():
            sums_ref[...] = jnp.zeros_like(sums_ref)
            cnts_ref[...] = jnp.zeros_like(cnts_ref)

        dinv = _dinv_of(degp_ref)
        pre = (q_ref[0] + q_ref[1] + hs2_ref[...]) * dinv + b2_ref[...]
        h2 = jnp.maximum(pre * (bng_ref[...] * _BNS) + bnb_ref[...], 0.0)
        oh = (batch_ref[...] ==
              lax.broadcasted_iota(jnp.int32, (1, _NG), 1)).astype(jnp.float32)
        sums_ref[...] += lax.dot_general(
            oh, h2, (((0,), (0,)), ((), ())),
            preferred_element_type=jnp.float32)
        cnts_ref[...] += lax.dot_general(
            oh, jnp.ones((_BM, 8), jnp.float32), (((0,), (0,)), ((), ())),
            preferred_element_type=jnp.float32)

        @pl.when(i == _G - 1)
        def _():
            xp = sums_ref[...] / jnp.maximum(cnts_ref[:, 0:1], 1.0)
            xb = xp * (bn1g_ref[...] * _BNS) + bn1b_ref[...]
            a = jnp.maximum(
                jnp.dot(xb, l1w_ref[...],
                        preferred_element_type=jnp.float32) + l1b_ref[...], 0.0)
            a = jnp.maximum(
                jnp.dot(a, l2w_ref[...],
                        preferred_element_type=jnp.float32) + l2b_ref[...], 0.0)
            a = jnp.maximum(
                jnp.dot(a, l3w_ref[...],
                        preferred_element_type=jnp.float32) + l3b_ref[...], 0.0)
            m = jnp.max(a, axis=1, keepdims=True)
            e = jnp.exp(a - m)
            out_ref[...] = e / jnp.sum(e, axis=1, keepdims=True)
            xp_ref[...] = xp

    full = lambda i: (0, 0)
    return pl.pallas_call(
        body,
        grid=(_G,),
        in_specs=[
            pl.BlockSpec((_NC, _BM, _H), lambda i: (0, i, 0)),
            pl.BlockSpec((_BM, _H), lambda i: (i, 0)),
            pl.BlockSpec((_NC, _BM, 16), lambda i: (0, i, 0)),
            pl.BlockSpec((_BM, 1), lambda i: (i, 0)),
            pl.BlockSpec((1, _H), full),
            pl.BlockSpec((1, _H), full),
            pl.BlockSpec((1, _H), full),
            pl.BlockSpec((1, _H), full),
            pl.BlockSpec((1, _H), full),
            pl.BlockSpec((_H, _DENSE), full),
            pl.BlockSpec((1, _DENSE), full),
            pl.BlockSpec((_DENSE, _DENSE), full),
            pl.BlockSpec((1, _DENSE), full),
            pl.BlockSpec((_DENSE, _NCLS), full),
            pl.BlockSpec((1, _NCLS), full),
        ],
        out_specs=[
            pl.BlockSpec((_NG, _NCLS), full),
            pl.BlockSpec((_NG, _H), full),
        ],
        out_shape=[
            jax.ShapeDtypeStruct((_NG, _NCLS), jnp.float32),
            jax.ShapeDtypeStruct((_NG, _H), jnp.float32),
        ],
        scratch_shapes=[
            pltpu.VMEM((_NG, _H), jnp.float32),
            pltpu.VMEM((_NG, 8), jnp.float32),
        ],
    )(q, hs2, degp, batch2, b2r, bngr, bnbr, bn1gr, bn1br,
      l1W, l1br, l2W, l2br, l3W, l3br)


def kernel(x, edge_index, batch, W1, b1, W2, b2, bng, bnb, bn1g, bn1b,
           l1W, l1b, l2W, l2b, l3W, l3b):
    src = edge_index[0].astype(jnp.int32)
    dst = edge_index[1].astype(jnp.int32)
    dst3 = dst.reshape(_NW, _CHD, _KD)
    packed3 = (src | (dst << 16)).reshape(_NW, _CH, _K)
    batch2 = batch.astype(jnp.int32).reshape(_N, 1)

    onesrow = jnp.concatenate(
        [jnp.ones((_KD, 1), jnp.float32), jnp.zeros((_KD, 15), jnp.float32)],
        axis=1)
    zrows = jnp.zeros((_NS, _RPT, 16), jnp.float32)
    zblk = jnp.zeros((_NS, _RPT, _H), jnp.float32)

    degp = _deg_partial(dst3, onesrow, zrows)
    hs1 = _mm_scale(x, W1, degp)
    p = _prop_partial(hs1, packed3, zblk)
    hs2 = _layer2(p, hs1, degp, b1.reshape(1, _H), W2)
    q = _prop_partial(hs2, packed3, zblk)
    out, xp = _final(
        q, hs2, degp, batch2,
        b2.reshape(1, _H), bng.reshape(1, _H), bnb.reshape(1, _H),
        bn1g.reshape(1, _H), bn1b.reshape(1, _H),
        l1W, l1b.reshape(1, _DENSE), l2W, l2b.reshape(1, _DENSE),
        l3W, l3b.reshape(1, _NCLS))
    return (out, xp)
